# baseline (speedup 1.0000x reference)
import jax
import jax.numpy as jnp
from jax import lax
from jax.experimental import pallas as pl
from jax.experimental.pallas import tpu as pltpu
from jax.experimental.pallas import tpu_sc as plsc

N = 10000
E = 320000
D = 128

NC = 2
NS = 16
NW = NC * NS
HC = 128
NCHUNK = E // HC
JPT = 79
NCHUNKP = JPT * NW
EP = NCHUNKP * HC
AC = 256
NACHUNK = E // AC
NPAD = 10240

RPT = 624
TAIL0 = NS * RPT
TAILN = N - TAIL0

RB = 1024
NGRID = NPAD // RB


def _head_math(x, ws_ref, wd_ref, wl_ref, ats_ref, atd_ref):
    hi = lax.Precision.HIGHEST
    ws = ws_ref[...]
    xs = lax.dot_general(x, ws, (((1,), (0,)), ((), ())), precision=hi,
                         preferred_element_type=jnp.float32)
    xl = lax.dot_general(x, wl_ref[...], (((1,), (0,)), ((), ())),
                         precision=hi, preferred_element_type=jnp.float32)
    u_s = lax.dot_general(ats_ref[...], ws, (((1,), (1,)), ((), ())),
                          precision=hi, preferred_element_type=jnp.float32)
    u_d = lax.dot_general(atd_ref[...], wd_ref[...], (((1,), (1,)), ((), ())),
                          precision=hi, preferred_element_type=jnp.float32)
    u2 = jnp.concatenate([u_s, u_d, jnp.zeros((6, D), jnp.float32)], axis=0)
    a8 = lax.dot_general(u2, x, (((1,), (1,)), ((), ())),
                         precision=hi, preferred_element_type=jnp.float32)
    return xs, xl, a8[0], a8[1]


def _gat_out_math(acc_ref, den_ref, xl_ref, bg_ref, bl_ref):
    ones = jnp.ones((NW, D), jnp.float32)
    dent = lax.dot_general(den_ref[...], ones, (((0,), (0,)), ((), ())),
                           precision=lax.Precision.HIGHEST,
                           preferred_element_type=jnp.float32)
    acc = acc_ref[0] + acc_ref[1]
    h = acc / (dent + 1e-16) + bg_ref[...] + xl_ref[...] + bl_ref[...]
    return jnp.maximum(h, 0.0)


_W_SPECS = [
    pl.BlockSpec((D, D), lambda i: (0, 0)),
    pl.BlockSpec((D, D), lambda i: (0, 0)),
    pl.BlockSpec((D, D), lambda i: (0, 0)),
    pl.BlockSpec((1, D), lambda i: (0, 0)),
    pl.BlockSpec((1, D), lambda i: (0, 0)),
]
_HEAD_OUT_SPECS = [
    pl.BlockSpec((RB, D), lambda i: (i, 0)),
    pl.BlockSpec((RB, D), lambda i: (i, 0)),
    pl.BlockSpec((RB,), lambda i: (i,)),
    pl.BlockSpec((RB,), lambda i: (i,)),
]
_HEAD_OUT_SHAPE = [
    jax.ShapeDtypeStruct((NPAD, D), jnp.float32),
    jax.ShapeDtypeStruct((NPAD, D), jnp.float32),
    jax.ShapeDtypeStruct((NPAD,), jnp.float32),
    jax.ShapeDtypeStruct((NPAD,), jnp.float32),
]
_GAT_IN_SPECS = [
    pl.BlockSpec((NC, RB, D), lambda i: (0, i, 0)),
    pl.BlockSpec((NW, RB), lambda i: (0, i)),
    pl.BlockSpec((RB, D), lambda i: (i, 0)),
    pl.BlockSpec((1, D), lambda i: (0, 0)),
    pl.BlockSpec((1, D), lambda i: (0, 0)),
]


def _tc_head_body(x_ref, ws_ref, wd_ref, wl_ref, ats_ref, atd_ref,
                  xs_ref, xl_ref, as_ref, ad_ref):
    xs, xl, a_s, a_d = _head_math(x_ref[...], ws_ref, wd_ref, wl_ref,
                                  ats_ref, atd_ref)
    xs_ref[...] = xs
    xl_ref[...] = xl
    as_ref[...] = a_s
    ad_ref[...] = a_d


_tc_head = pl.pallas_call(
    _tc_head_body,
    grid=(NGRID,),
    in_specs=[pl.BlockSpec((RB, D), lambda i: (i, 0))] + _W_SPECS,
    out_specs=_HEAD_OUT_SPECS,
    out_shape=_HEAD_OUT_SHAPE,
)


def _tc_mid_body(acc_ref, den_ref, xl_ref, bg_ref, bl_ref,
                 ws_ref, wd_ref, wl_ref, ats_ref, atd_ref,
                 xs_ref, xl2_ref, as_ref, ad_ref):
    h = _gat_out_math(acc_ref, den_ref, xl_ref, bg_ref, bl_ref)
    xs, xl2, a_s, a_d = _head_math(h, ws_ref, wd_ref, wl_ref,
                                   ats_ref, atd_ref)
    xs_ref[...] = xs
    xl2_ref[...] = xl2
    as_ref[...] = a_s
    ad_ref[...] = a_d


_tc_mid = pl.pallas_call(
    _tc_mid_body,
    grid=(NGRID,),
    in_specs=_GAT_IN_SPECS + _W_SPECS,
    out_specs=_HEAD_OUT_SPECS,
    out_shape=_HEAD_OUT_SHAPE,
)


def _tc_epi_body(acc_ref, den_ref, xl_ref, bg_ref, bl_ref, out_ref):
    out_ref[...] = _gat_out_math(acc_ref, den_ref, xl_ref, bg_ref, bl_ref)


_tc_epi = pl.pallas_call(
    _tc_epi_body,
    grid=(NGRID,),
    in_specs=_GAT_IN_SPECS,
    out_specs=pl.BlockSpec((RB, D), lambda i: (i, 0)),
    out_shape=jax.ShapeDtypeStruct((NPAD, D), jnp.float32),
)


def _sc_alpha_body(src_ref, dst_ref, as_ref, ad_ref, ex_out, den_out,
                   a_s_v, a_d_v, den_v, idxB, exb0, exb1,
                   semi0, semi1, semx0, semx1):
    c = lax.axis_index("c")
    s = lax.axis_index("s")
    w = s * NC + c

    exb = [exb0, exb1]
    semi = [semi0, semi1]
    semx = [semx0, semx1]

    pltpu.sync_copy(as_ref, a_s_v)
    pltpu.sync_copy(ad_ref, a_d_v)

    zeros16 = jnp.zeros((16,), jnp.float32)

    def _zero_den(i, _):
        den_v[pl.ds(i * 16, 16)] = zeros16
        return 0
    lax.fori_loop(0, NPAD // 16, _zero_den, 0)

    def _issue_idx(cidx, p):
        pltpu.async_copy(src_ref.at[cidx], idxB.at[pl.ds(2 * p, 2)], semi[p])
        pltpu.async_copy(dst_ref.at[cidx], idxB.at[pl.ds(4 + 2 * p, 2)],
                         semi[p])

    def _wait_idx(cidx, p):
        pltpu.make_async_copy(src_ref.at[cidx], idxB.at[pl.ds(2 * p, 2)],
                              semi[p]).wait()
        pltpu.make_async_copy(dst_ref.at[cidx], idxB.at[pl.ds(4 + 2 * p, 2)],
                              semi[p]).wait()

    def _scalar_phase(p):
        for h in range(2):
            for gg in range(HC // 16):
                sl = pl.ds(gg * 16, 16)
                sv = idxB[2 * p + h, sl]
                dv = idxB[4 + 2 * p + h, sl]
                a = (plsc.load_gather(a_s_v, [sv])
                     + plsc.load_gather(a_d_v, [dv]))
                a = jnp.maximum(a, 0.2 * a)
                e = jnp.exp(a)
                exb[p][pl.ds(h * HC + gg * 16, 16)] = e
                plsc.addupdate_scatter(den_v, [dv], e)

    def _issue_ex(cidx, p):
        pltpu.async_copy(exb[p], ex_out.at[pl.ds(cidx * AC, AC)], semx[p])

    def _wait_ex(cidx, p):
        pltpu.make_async_copy(exb[p], ex_out.at[pl.ds(cidx * AC, AC)],
                              semx[p]).wait()

    _issue_idx(w, 0)

    def _pipe(i, _):
        for t in range(2):
            p = t
            q = 1 - t
            c_cur = (2 * i + t) * NW + w
            c_nxt = c_cur + NW

            @pl.when(c_nxt < NACHUNK)
            def _():
                _issue_idx(c_nxt, q)

            @pl.when(c_cur < NACHUNK)
            def _():
                _wait_idx(c_cur, p)

                @pl.when(i > 0)
                def _():
                    _wait_ex(c_cur - 2 * NW, p)
                _scalar_phase(p)
                _issue_ex(c_cur, p)
        return 0

    lax.fori_loop(0, 21, _pipe, 0)

    _wait_ex(0, 0)
    _wait_ex(0, 1)

    @pl.when(w == 0)
    def _():
        def _zero_ex(i, _):
            exb0[pl.ds(i * 16, 16)] = zeros16
            return 0
        lax.fori_loop(0, AC // 16, _zero_ex, 0)
        for k in range((EP - E) // AC):
            pltpu.sync_copy(exb0, ex_out.at[pl.ds(E + k * AC, AC)])

    pltpu.sync_copy(den_v, den_out.at[pl.ds(w * NPAD, NPAD)])


_sc_alpha = pl.kernel(
    _sc_alpha_body,
    out_type=[
        jax.ShapeDtypeStruct((EP,), jnp.float32),
        jax.ShapeDtypeStruct((NW * NPAD,), jnp.float32),
    ],
    mesh=plsc.VectorSubcoreMesh(core_axis_name="c", subcore_axis_name="s"),
    compiler_params=pltpu.CompilerParams(needs_layout_passes=False),
    scratch_types=[
        pltpu.VMEM((NPAD,), jnp.float32),
        pltpu.VMEM((NPAD,), jnp.float32),
        pltpu.VMEM((NPAD,), jnp.float32),
        pltpu.VMEM((8, HC), jnp.int32),
        pltpu.VMEM((AC,), jnp.float32),
        pltpu.VMEM((AC,), jnp.float32),
        pltpu.SemaphoreType.DMA,
        pltpu.SemaphoreType.DMA,
        pltpu.SemaphoreType.DMA,
        pltpu.SemaphoreType.DMA,
    ],
)


def _sc_rows_body(src_ref, dst_ref, ex_ref, xs_ref, acc_out,
                  idxB, exb0, exb1, exb2, rows0, rows1, rows2,
                  semi0, semi1, semi2, semg0, semg1, semg2,
                  semsc0, semsc1, semsc2,
                  acc_sh):
    c = lax.axis_index("c")
    s = lax.axis_index("s")
    w = s * NC + c

    exb = [exb0, exb1, exb2]
    semi = [semi0, semi1, semi2]
    rows = [rows0, rows1, rows2]
    semg = [semg0, semg1, semg2]
    semsc = [semsc0, semsc1, semsc2]

    zeros16 = jnp.zeros((16,), jnp.float32)

    def _zero_rows(i, _):
        for cc in range(D // 16):
            rows0[i, pl.ds(cc * 16, 16)] = zeros16
        return 0
    lax.fori_loop(0, 104, _zero_rows, 0)
    for j in range(RPT // 104):
        pltpu.sync_copy(rows0.at[pl.ds(0, 104)],
                        acc_sh.at[pl.ds(s * RPT + j * 104, 104)])

    @pl.when(s == NS - 1)
    def _():
        pltpu.sync_copy(rows0.at[pl.ds(0, TAILN)],
                        acc_sh.at[pl.ds(TAIL0, TAILN)])

    plsc.subcore_barrier()


    def _issue_idx(cidx, g):
        pltpu.async_copy(src_ref.at[cidx], idxB.at[pl.ds(g, 1)], semi[g])
        pltpu.async_copy(dst_ref.at[cidx], idxB.at[pl.ds(3 + g, 1)], semi[g])
        pltpu.async_copy(ex_ref.at[pl.ds(cidx * HC, HC)], exb[g], semi[g])

    def _wait_idx(cidx, g):
        pltpu.make_async_copy(src_ref.at[cidx], idxB.at[pl.ds(g, 1)],
                              semi[g]).wait()
        pltpu.make_async_copy(dst_ref.at[cidx], idxB.at[pl.ds(3 + g, 1)],
                              semi[g]).wait()
        pltpu.make_async_copy(ex_ref.at[pl.ds(cidx * HC, HC)], exb[g],
                              semi[g]).wait()

    def _issue_gather(g):
        pltpu.async_copy(xs_ref.at[idxB.at[g]], rows[g], semg[g])

    def _wait_gather(g):
        pltpu.make_async_copy(xs_ref.at[idxB.at[g]], rows[g],
                              semg[g]).wait()

    def _issue_scatter(g):
        pltpu.async_copy(rows[g], acc_sh.at[idxB.at[3 + g]], semsc[g],
                         add=True)

    def _wait_scatter(g):
        pltpu.make_async_copy(rows[g], acc_sh.at[idxB.at[3 + g]],
                              semsc[g]).wait()

    def _scale(g):
        def body(gg, _):
            ev16 = exb[g][pl.ds(gg * 16, 16)]
            r0 = gg * 16
            for jj in range(16):
                ev = jnp.broadcast_to(ev16[jj], (16,))
                for cc in range(D // 16):
                    sl = pl.ds(cc * 16, 16)
                    rows[g][r0 + jj, sl] = rows[g][r0 + jj, sl] * ev
            return 0
        lax.fori_loop(0, HC // 16, body, 0)

    _issue_idx(w, 0)
    _wait_idx(w, 0)
    _issue_gather(0)
    _issue_idx(NW + w, 1)

    def _pipe(i, _):
        for t in range(6):
            g_cur = t % 3
            g_nxt = (t + 1) % 3
            g_nn = (t + 2) % 3
            c_cur = (6 * i + t) * NW + w
            c_nxt = c_cur + NW
            c_nn = c_cur + 2 * NW

            _wait_idx(c_nxt, g_nxt)
            if t < 2:
                @pl.when(i > 0)
                def _():
                    _wait_scatter(g_nxt)
            else:
                _wait_scatter(g_nxt)
            _issue_gather(g_nxt)

            @pl.when(c_nn < NCHUNKP)
            def _():
                _issue_idx(c_nn, g_nn)

            _wait_gather(g_cur)
            _scale(g_cur)
            _issue_scatter(g_cur)
        return 0

    lax.fori_loop(0, (JPT - 1) // 6, _pipe, 0)

    _wait_gather(0)
    _scale(0)
    _issue_scatter(0)
    _wait_scatter(1)
    _wait_scatter(2)
    _wait_scatter(0)

    plsc.subcore_barrier()

    row0 = s * RPT
    pltpu.sync_copy(acc_sh.at[pl.ds(row0, RPT)],
                    acc_out.at[c, pl.ds(row0, RPT)])

    @pl.when(s == NS - 1)
    def _():
        pltpu.sync_copy(acc_sh.at[pl.ds(TAIL0, TAILN)],
                        acc_out.at[c, pl.ds(TAIL0, TAILN)])

    @pl.when(s == 0)
    def _():
        def _zero_pad(i, _):
            for cc in range(D // 16):
                rows0[i, pl.ds(cc * 16, 16)] = zeros16
            return 0
        lax.fori_loop(0, 120, _zero_pad, 0)
        pltpu.sync_copy(rows0.at[pl.ds(0, 120)],
                        acc_out.at[c, pl.ds(N, 120)])
        pltpu.sync_copy(rows0.at[pl.ds(0, 120)],
                        acc_out.at[c, pl.ds(N + 120, 120)])


_sc_rows = pl.kernel(
    _sc_rows_body,
    out_type=jax.ShapeDtypeStruct((NC, NPAD, D), jnp.float32),
    mesh=plsc.VectorSubcoreMesh(core_axis_name="c", subcore_axis_name="s"),
    compiler_params=pltpu.CompilerParams(needs_layout_passes=False),
    scratch_types=[
        pltpu.VMEM((6, HC), jnp.int32),
        pltpu.VMEM((HC,), jnp.float32),
        pltpu.VMEM((HC,), jnp.float32),
        pltpu.VMEM((HC,), jnp.float32),
        pltpu.VMEM((HC, D), jnp.float32),
        pltpu.VMEM((HC, D), jnp.float32),
        pltpu.VMEM((HC, D), jnp.float32),
        pltpu.SemaphoreType.DMA,
        pltpu.SemaphoreType.DMA,
        pltpu.SemaphoreType.DMA,
        pltpu.SemaphoreType.DMA,
        pltpu.SemaphoreType.DMA,
        pltpu.SemaphoreType.DMA,
        pltpu.SemaphoreType.DMA,
        pltpu.SemaphoreType.DMA,
        pltpu.SemaphoreType.DMA,
        pltpu.VMEM_SHARED((N, D), jnp.float32),
    ],
)


def kernel(x, edge_index, W_src0, W_dst0, att_src0, att_dst0, b_gat0,
           W_lin0, b_lin0, W_src1, W_dst1, att_src1, att_dst1, b_gat1,
           W_lin1, b_lin1):
    x = x.astype(jnp.float32)
    ei = edge_index.astype(jnp.int32)
    src_a = ei[0].reshape(NACHUNK, 2, HC)
    dst_a = ei[1].reshape(NACHUNK, 2, HC)
    src_r = jnp.concatenate([ei[0], ei[0][:EP - E]]).reshape(NCHUNKP, 1, HC)
    dst_r = jnp.concatenate([ei[1], ei[1][:EP - E]]).reshape(NCHUNKP, 1, HC)
    xp = jnp.pad(x, ((0, NPAD - N), (0, 0)))
    xs, xl, a_s, a_d = _tc_head(xp, W_src0, W_dst0, W_lin0,
                                att_src0.reshape(1, D),
                                att_dst0.reshape(1, D))
    exh, den = _sc_alpha(src_a, dst_a, a_s, a_d)
    acc = _sc_rows(src_r, dst_r, exh, xs)
    xs, xl, a_s, a_d = _tc_mid(acc, den.reshape(NW, NPAD), xl,
                               b_gat0.reshape(1, D), b_lin0.reshape(1, D),
                               W_src1, W_dst1, W_lin1,
                               att_src1.reshape(1, D),
                               att_dst1.reshape(1, D))
    exh, den = _sc_alpha(src_a, dst_a, a_s, a_d)
    acc = _sc_rows(src_r, dst_r, exh, xs)
    hp = _tc_epi(acc, den.reshape(NW, NPAD), xl,
                 b_gat1.reshape(1, D), b_lin1.reshape(1, D))
    return hp[:N]


# speedup vs baseline: 1.1035x; 1.1035x over previous
import jax
import jax.numpy as jnp
from jax import lax
from jax.experimental import pallas as pl
from jax.experimental.pallas import tpu as pltpu
from jax.experimental.pallas import tpu_sc as plsc

N = 10000
E = 320000
D = 128

NC = 2
NS = 16
NW = NC * NS
HC = 128
NCHUNK = E // HC
JPT = 79
NCHUNKP = JPT * NW
EP = NCHUNKP * HC
AC = 512
NACHUNK = E // AC
NPAD = 10112

RPT = 624
TAIL0 = NS * RPT
TAILN = N - TAIL0

RB = 1000


def _tc_pre_body(x_ref, ws_ref, wl_ref, xs_ref, xl_ref):
    x = x_ref[...]
    hi = lax.Precision.HIGHEST
    xs_ref[...] = lax.dot_general(x, ws_ref[...], (((1,), (0,)), ((), ())),
                                  precision=hi,
                                  preferred_element_type=jnp.float32)
    xl_ref[...] = lax.dot_general(x, wl_ref[...], (((1,), (0,)), ((), ())),
                                  precision=hi,
                                  preferred_element_type=jnp.float32)


_tc_pre = pl.pallas_call(
    _tc_pre_body,
    grid=(N // RB,),
    in_specs=[
        pl.BlockSpec((RB, D), lambda i: (i, 0)),
        pl.BlockSpec((D, D), lambda i: (0, 0)),
        pl.BlockSpec((D, D), lambda i: (0, 0)),
    ],
    out_specs=[
        pl.BlockSpec((RB, D), lambda i: (i, 0)),
        pl.BlockSpec((RB, D), lambda i: (i, 0)),
    ],
    out_shape=[
        jax.ShapeDtypeStruct((N, D), jnp.float32),
        jax.ShapeDtypeStruct((N, D), jnp.float32),
    ],
)


def _tc_att_body(x_ref, ws_ref, wd_ref, ats_ref, atd_ref, as_ref, ad_ref):
    hi = lax.Precision.HIGHEST
    u_s = lax.dot_general(ats_ref[...], ws_ref[...], (((1,), (1,)), ((), ())),
                          precision=hi, preferred_element_type=jnp.float32)
    u_d = lax.dot_general(atd_ref[...], wd_ref[...], (((1,), (1,)), ((), ())),
                          precision=hi, preferred_element_type=jnp.float32)
    u2 = jnp.concatenate([u_s, u_d, jnp.zeros((6, D), jnp.float32)], axis=0)
    a8 = lax.dot_general(u2, x_ref[...], (((1,), (1,)), ((), ())),
                         precision=hi, preferred_element_type=jnp.float32)
    as_ref[...] = a8[0]
    ad_ref[...] = a8[1]


_tc_att = pl.pallas_call(
    _tc_att_body,
    out_shape=[
        jax.ShapeDtypeStruct((N,), jnp.float32),
        jax.ShapeDtypeStruct((N,), jnp.float32),
    ],
)


def _tc_den_body(den_ref, dent_ref):
    ones = jnp.ones((NW, D), jnp.float32)
    dent_ref[...] = lax.dot_general(den_ref[...], ones,
                                    (((0,), (0,)), ((), ())),
                                    precision=lax.Precision.HIGHEST,
                                    preferred_element_type=jnp.float32)


_tc_den = pl.pallas_call(
    _tc_den_body,
    out_shape=jax.ShapeDtypeStruct((NPAD, D), jnp.float32),
)


def _tc_epi_body(acc_ref, dent_ref, xl_ref, bg_ref, bl_ref, out_ref):
    acc = acc_ref[0] + acc_ref[1]
    h = (acc / (dent_ref[...] + 1e-16) + bg_ref[...] + xl_ref[...]
         + bl_ref[...])
    out_ref[...] = jnp.maximum(h, 0.0)


_tc_epi = pl.pallas_call(
    _tc_epi_body,
    grid=(N // RB,),
    in_specs=[
        pl.BlockSpec((NC, RB, D), lambda i: (0, i, 0)),
        pl.BlockSpec((RB, D), lambda i: (i, 0)),
        pl.BlockSpec((RB, D), lambda i: (i, 0)),
        pl.BlockSpec((1, D), lambda i: (0, 0)),
        pl.BlockSpec((1, D), lambda i: (0, 0)),
    ],
    out_specs=pl.BlockSpec((RB, D), lambda i: (i, 0)),
    out_shape=jax.ShapeDtypeStruct((N, D), jnp.float32),
)


def _sc_alpha_body(src_ref, dst_ref, as_ref, ad_ref, ex_out, den_out,
                   a_s_v, a_d_v, den_v, idxB, exb0, exb1,
                   semi0, semi1, semx0, semx1):
    c = lax.axis_index("c")
    s = lax.axis_index("s")
    w = s * NC + c

    exb = [exb0, exb1]
    semi = [semi0, semi1]
    semx = [semx0, semx1]

    pltpu.sync_copy(as_ref, a_s_v)
    pltpu.sync_copy(ad_ref, a_d_v)

    zeros16 = jnp.zeros((16,), jnp.float32)

    def _zero_den(i, _):
        den_v[pl.ds(i * 16, 16)] = zeros16
        return 0
    lax.fori_loop(0, NPAD // 16, _zero_den, 0)

    def _issue_idx(cidx, p):
        pltpu.async_copy(src_ref.at[cidx], idxB.at[pl.ds(4 * p, 4)], semi[p])
        pltpu.async_copy(dst_ref.at[cidx], idxB.at[pl.ds(8 + 4 * p, 4)],
                         semi[p])

    def _wait_idx(cidx, p):
        pltpu.make_async_copy(src_ref.at[cidx], idxB.at[pl.ds(4 * p, 4)],
                              semi[p]).wait()
        pltpu.make_async_copy(dst_ref.at[cidx], idxB.at[pl.ds(8 + 4 * p, 4)],
                              semi[p]).wait()

    def _scalar_phase(p):
        for h in range(4):
            for gg in range(HC // 16):
                sl = pl.ds(gg * 16, 16)
                sv = idxB[4 * p + h, sl]
                dv = idxB[8 + 4 * p + h, sl]
                a = (plsc.load_gather(a_s_v, [sv])
                     + plsc.load_gather(a_d_v, [dv]))
                a = jnp.maximum(a, 0.2 * a)
                e = jnp.exp(a)
                exb[p][pl.ds(h * HC + gg * 16, 16)] = e
                plsc.addupdate_scatter(den_v, [dv], e)

    def _issue_ex(cidx, p):
        pltpu.async_copy(exb[p], ex_out.at[pl.ds(cidx * AC, AC)], semx[p])

    def _wait_ex(cidx, p):
        pltpu.make_async_copy(exb[p], ex_out.at[pl.ds(cidx * AC, AC)],
                              semx[p]).wait()

    _issue_idx(w, 0)

    def _pipe(i, _):
        for t in range(2):
            p = t
            q = 1 - t
            c_cur = (2 * i + t) * NW + w
            c_nxt = c_cur + NW

            @pl.when(c_nxt < NACHUNK)
            def _():
                _issue_idx(c_nxt, q)

            @pl.when(c_cur < NACHUNK)
            def _():
                _wait_idx(c_cur, p)

                @pl.when(i > 0)
                def _():
                    _wait_ex(c_cur - 2 * NW, p)
                _scalar_phase(p)
                _issue_ex(c_cur, p)
        return 0

    lax.fori_loop(0, 11, _pipe, 0)

    _wait_ex(0, 0)
    _wait_ex(0, 1)

    @pl.when(w == 0)
    def _():
        def _zero_ex(i, _):
            exb0[pl.ds(i * 16, 16)] = zeros16
            return 0
        lax.fori_loop(0, AC // 16, _zero_ex, 0)
        for k in range((EP - E) // AC):
            pltpu.sync_copy(exb0, ex_out.at[pl.ds(E + k * AC, AC)])

    pltpu.sync_copy(den_v, den_out.at[pl.ds(w * NPAD, NPAD)])


_sc_alpha = pl.kernel(
    _sc_alpha_body,
    out_type=[
        jax.ShapeDtypeStruct((EP,), jnp.float32),
        jax.ShapeDtypeStruct((NW * NPAD,), jnp.float32),
    ],
    mesh=plsc.VectorSubcoreMesh(core_axis_name="c", subcore_axis_name="s"),
    compiler_params=pltpu.CompilerParams(needs_layout_passes=False),
    scratch_types=[
        pltpu.VMEM((N,), jnp.float32),
        pltpu.VMEM((N,), jnp.float32),
        pltpu.VMEM((NPAD,), jnp.float32),
        pltpu.VMEM((16, HC), jnp.int32),
        pltpu.VMEM((AC,), jnp.float32),
        pltpu.VMEM((AC,), jnp.float32),
        pltpu.SemaphoreType.DMA,
        pltpu.SemaphoreType.DMA,
        pltpu.SemaphoreType.DMA,
        pltpu.SemaphoreType.DMA,
    ],
)


def _sc_rows_body(src_ref, dst_ref, ex_ref, xs_ref, acc_out,
                  idxB, exb0, exb1, exb2, rows0, rows1, rows2,
                  semi0, semi1, semi2, semg0, semg1, semg2,
                  semsc0, semsc1, semsc2,
                  acc_sh):
    c = lax.axis_index("c")
    s = lax.axis_index("s")
    w = s * NC + c

    exb = [exb0, exb1, exb2]
    semi = [semi0, semi1, semi2]
    rows = [rows0, rows1, rows2]
    semg = [semg0, semg1, semg2]
    semsc = [semsc0, semsc1, semsc2]

    zeros16 = jnp.zeros((16,), jnp.float32)

    def _zero_rows(i, _):
        for cc in range(D // 16):
            rows0[i, pl.ds(cc * 16, 16)] = zeros16
        return 0
    lax.fori_loop(0, 104, _zero_rows, 0)
    for j in range(RPT // 104):
        pltpu.sync_copy(rows0.at[pl.ds(0, 104)],
                        acc_sh.at[pl.ds(s * RPT + j * 104, 104)])

    @pl.when(s == NS - 1)
    def _():
        pltpu.sync_copy(rows0.at[pl.ds(0, TAILN)],
                        acc_sh.at[pl.ds(TAIL0, TAILN)])

    plsc.subcore_barrier()


    def _issue_idx(cidx, g):
        pltpu.async_copy(src_ref.at[cidx], idxB.at[pl.ds(g, 1)], semi[g])
        pltpu.async_copy(dst_ref.at[cidx], idxB.at[pl.ds(3 + g, 1)], semi[g])
        pltpu.async_copy(ex_ref.at[pl.ds(cidx * HC, HC)], exb[g], semi[g])

    def _wait_idx(cidx, g):
        pltpu.make_async_copy(src_ref.at[cidx], idxB.at[pl.ds(g, 1)],
                              semi[g]).wait()
        pltpu.make_async_copy(dst_ref.at[cidx], idxB.at[pl.ds(3 + g, 1)],
                              semi[g]).wait()
        pltpu.make_async_copy(ex_ref.at[pl.ds(cidx * HC, HC)], exb[g],
                              semi[g]).wait()

    def _issue_gather(g):
        pltpu.async_copy(xs_ref.at[idxB.at[g]], rows[g], semg[g])

    def _wait_gather(g):
        pltpu.make_async_copy(xs_ref.at[idxB.at[g]], rows[g],
                              semg[g]).wait()

    def _issue_scatter(g):
        pltpu.async_copy(rows[g], acc_sh.at[idxB.at[3 + g]], semsc[g],
                         add=True)

    def _wait_scatter(g):
        pltpu.make_async_copy(rows[g], acc_sh.at[idxB.at[3 + g]],
                              semsc[g]).wait()

    def _scale(g):
        def body(gg, _):
            ev16 = exb[g][pl.ds(gg * 16, 16)]
            r0 = gg * 16
            for jj in range(16):
                ev = jnp.broadcast_to(ev16[jj], (16,))
                for cc in range(D // 16):
                    sl = pl.ds(cc * 16, 16)
                    rows[g][r0 + jj, sl] = rows[g][r0 + jj, sl] * ev
            return 0
        lax.fori_loop(0, HC // 16, body, 0)

    _issue_idx(w, 0)
    _wait_idx(w, 0)
    _issue_gather(0)
    _issue_idx(NW + w, 1)

    def _pipe(i, _):
        for t in range(6):
            g_cur = t % 3
            g_nxt = (t + 1) % 3
            g_nn = (t + 2) % 3
            c_cur = (6 * i + t) * NW + w
            c_nxt = c_cur + NW
            c_nn = c_cur + 2 * NW

            _wait_idx(c_nxt, g_nxt)
            if t < 2:
                @pl.when(i > 0)
                def _():
                    _wait_scatter(g_nxt)
            else:
                _wait_scatter(g_nxt)
            _issue_gather(g_nxt)

            @pl.when(c_nn < NCHUNKP)
            def _():
                _issue_idx(c_nn, g_nn)

            _wait_gather(g_cur)
            _scale(g_cur)
            _issue_scatter(g_cur)
        return 0

    lax.fori_loop(0, (JPT - 1) // 6, _pipe, 0)

    _wait_gather(0)
    _scale(0)
    _issue_scatter(0)
    _wait_scatter(1)
    _wait_scatter(2)
    _wait_scatter(0)

    plsc.subcore_barrier()

    row0 = s * RPT
    pltpu.sync_copy(acc_sh.at[pl.ds(row0, RPT)],
                    acc_out.at[c, pl.ds(row0, RPT)])

    @pl.when(s == NS - 1)
    def _():
        pltpu.sync_copy(acc_sh.at[pl.ds(TAIL0, TAILN)],
                        acc_out.at[c, pl.ds(TAIL0, TAILN)])


_sc_rows = pl.kernel(
    _sc_rows_body,
    out_type=jax.ShapeDtypeStruct((NC, N, D), jnp.float32),
    mesh=plsc.VectorSubcoreMesh(core_axis_name="c", subcore_axis_name="s"),
    compiler_params=pltpu.CompilerParams(needs_layout_passes=False),
    scratch_types=[
        pltpu.VMEM((6, HC), jnp.int32),
        pltpu.VMEM((HC,), jnp.float32),
        pltpu.VMEM((HC,), jnp.float32),
        pltpu.VMEM((HC,), jnp.float32),
        pltpu.VMEM((HC, D), jnp.float32),
        pltpu.VMEM((HC, D), jnp.float32),
        pltpu.VMEM((HC, D), jnp.float32),
        pltpu.SemaphoreType.DMA,
        pltpu.SemaphoreType.DMA,
        pltpu.SemaphoreType.DMA,
        pltpu.SemaphoreType.DMA,
        pltpu.SemaphoreType.DMA,
        pltpu.SemaphoreType.DMA,
        pltpu.SemaphoreType.DMA,
        pltpu.SemaphoreType.DMA,
        pltpu.SemaphoreType.DMA,
        pltpu.VMEM_SHARED((N, D), jnp.float32),
    ],
)


def _layer(h, src_a, dst_a, src_r, dst_r, W_src, W_dst, att_src, att_dst,
           b_gat, W_lin, b_lin):
    xs, xl = _tc_pre(h, W_src, W_lin)
    a_s, a_d = _tc_att(h, W_src, W_dst, att_src.reshape(1, D),
                       att_dst.reshape(1, D))
    exh, den = _sc_alpha(src_a, dst_a, a_s, a_d)
    acc = _sc_rows(src_r, dst_r, exh, xs)
    dent = _tc_den(den.reshape(NW, NPAD))
    return _tc_epi(acc, dent, xl, b_gat.reshape(1, D), b_lin.reshape(1, D))


def kernel(x, edge_index, W_src0, W_dst0, att_src0, att_dst0, b_gat0,
           W_lin0, b_lin0, W_src1, W_dst1, att_src1, att_dst1, b_gat1,
           W_lin1, b_lin1):
    x = x.astype(jnp.float32)
    ei = edge_index.astype(jnp.int32)
    src_a = ei[0].reshape(NACHUNK, 4, HC)
    dst_a = ei[1].reshape(NACHUNK, 4, HC)
    src_r = jnp.concatenate([ei[0], ei[0][:EP - E]]).reshape(NCHUNKP, 1, HC)
    dst_r = jnp.concatenate([ei[1], ei[1][:EP - E]]).reshape(NCHUNKP, 1, HC)
    h = _layer(x, src_a, dst_a, src_r, dst_r, W_src0, W_dst0, att_src0,
               att_dst0, b_gat0, W_lin0, b_lin0)
    h = _layer(h, src_a, dst_a, src_r, dst_r, W_src1, W_dst1, att_src1,
               att_dst1, b_gat1, W_lin1, b_lin1)
    return h
